# baseline (speedup 1.0000x reference)
import functools

import jax
import jax.numpy as jnp
from jax import lax
from jax.experimental import pallas as pl
from jax.experimental.pallas import tpu as pltpu
from jax.experimental.pallas import tpu_sc as plsc

_NP = 4096
_P = 64

_BM = 256
_NTC = 12 * _BM

_L = 16
_NC = 2
_NS = 16
_NW = _NC * _NS
_NSC = _NP - _NTC
_PPW = _NSC // _NW

_BIG = 3.0e38

_DN = (((1,), (1,)), ((0,), (0,)))


def _chamfer_body(pred_ref, tgt_ref, out_ref):
    @pl.when(pl.program_id(0) == 0)
    def _init():
        out_ref[...] = jnp.zeros_like(out_ref)

    p = pred_ref[...]
    t = tgt_ref[...]
    pn = jnp.sum(p * p, axis=1, keepdims=True)
    tn = jnp.sum(t * t, axis=1, keepdims=True)
    ones = jnp.ones_like(pn)

    lhs = jnp.concatenate([p, pn, ones], axis=1)
    rhs = jnp.concatenate([-2.0 * t, ones, tn], axis=1)

    d2 = jax.lax.dot_general(lhs, rhs, _DN, preferred_element_type=jnp.float32)

    fwd = jnp.min(d2, axis=2)
    bwd = jnp.min(d2, axis=1)
    step = jnp.sum(fwd) + jnp.sum(bwd)
    out_ref[...] += step.reshape(1, 1)


def _dir_min_sum(a_ref, b_ref, i, acc):
    ax = [a_ref[i, 0, pl.ds(c * _L, _L)] for c in range(4)]
    ay = [a_ref[i, 1, pl.ds(c * _L, _L)] for c in range(4)]
    az = [a_ref[i, 2, pl.ds(c * _L, _L)] for c in range(4)]
    an = [ax[c] * ax[c] + ay[c] * ay[c] + az[c] * az[c] for c in range(4)]

    big = jnp.full((_L,), _BIG, jnp.float32)
    mins = [[big] * 4, [big] * 4]

    for qc in range(4):
        bx = b_ref[i, 0, pl.ds(qc * _L, _L)]
        by = b_ref[i, 1, pl.ds(qc * _L, _L)]
        bz = b_ref[i, 2, pl.ds(qc * _L, _L)]
        bn = bx * bx + by * by + bz * bz
        bm2x = -2.0 * bx
        bm2y = -2.0 * by
        bm2z = -2.0 * bz
        m = mins[qc & 1]
        for l in range(_L):
            sx, sy, sz, sn = bm2x[l], bm2y[l], bm2z[l], bn[l]
            for c in range(4):
                d2 = (an[c] + sn) + sx * ax[c] + sy * ay[c] + sz * az[c]
                m[c] = jnp.minimum(m[c], d2)

    tot = acc
    for c in range(4):
        tot = tot + jnp.maximum(jnp.minimum(mins[0][c], mins[1][c]), 0.0)
    return tot


def _sc_chamfer(pred_hbm, tgt_hbm, out_hbm, pred_v, tgt_v, acc_v):
    wid = lax.axis_index("s") * _NC + lax.axis_index("c")
    base = wid * _PPW
    pltpu.sync_copy(pred_hbm.at[pl.ds(base, _PPW)], pred_v)
    pltpu.sync_copy(tgt_hbm.at[pl.ds(base, _PPW)], tgt_v)

    def patch_body(i, acc):
        acc = _dir_min_sum(pred_v, tgt_v, i, acc)
        acc = _dir_min_sum(tgt_v, pred_v, i, acc)
        return acc

    acc = lax.fori_loop(0, _PPW, patch_body, jnp.zeros((_L,), jnp.float32))
    acc_v[...] = acc
    pltpu.sync_copy(acc_v, out_hbm.at[wid])


def kernel(pred_patches, target_patches):
    pred_raw = pred_patches.reshape(_NP, _P, 3)
    tgt_raw = target_patches.reshape(_NP, _P, 3)
    sc_pred = pred_raw[_NTC:].swapaxes(1, 2)
    sc_tgt = tgt_raw[_NTC:].swapaxes(1, 2)
    pred = pred_raw[:_NTC].swapaxes(1, 2)
    tgt = tgt_raw[:_NTC].swapaxes(1, 2)

    mesh = plsc.VectorSubcoreMesh(core_axis_name="c", subcore_axis_name="s")
    sc_run = functools.partial(
        pl.kernel,
        mesh=mesh,
        out_type=jax.ShapeDtypeStruct((_NW, _L), jnp.float32),
        scratch_types=[
            pltpu.VMEM((_PPW, 3, _P), jnp.float32),
            pltpu.VMEM((_PPW, 3, _P), jnp.float32),
            pltpu.VMEM((_L,), jnp.float32),
        ],
    )(_sc_chamfer)
    sc_partial = sc_run(sc_pred, sc_tgt)

    raw = pl.BlockSpec((_BM, 3, _P), lambda i: (i, 0, 0))
    tc_total = pl.pallas_call(
        _chamfer_body,
        grid=(_NTC // _BM,),
        in_specs=[raw, raw],
        out_specs=pl.BlockSpec((1, 1), lambda i: (0, 0)),
        out_shape=jax.ShapeDtypeStruct((1, 1), jnp.float32),
    )(pred, tgt)

    return (tc_total[0, 0] + jnp.sum(sc_partial)) * (1.0 / (_NP * _P))


# speedup vs baseline: 1.1668x; 1.1668x over previous
import functools

import jax
import jax.numpy as jnp
from jax import lax
from jax.experimental import pallas as pl
from jax.experimental.pallas import tpu as pltpu
from jax.experimental.pallas import tpu_sc as plsc

_NP = 4096
_P = 64

_BM = 212
_NTC = 16 * _BM

_L = 16
_NC = 2
_NS = 16
_NW = _NC * _NS
_NSC = _NP - _NTC
_PPW = _NSC // _NW

_BIG = 3.0e38

_DN = (((1,), (1,)), ((0,), (0,)))


def _chamfer_body(pred_ref, tgt_ref, out_ref):
    @pl.when(pl.program_id(0) == 0)
    def _init():
        out_ref[...] = jnp.zeros_like(out_ref)

    p = pred_ref[...]
    t = tgt_ref[...]
    pn = jnp.sum(p * p, axis=1, keepdims=True)
    tn = jnp.sum(t * t, axis=1, keepdims=True)
    ones = jnp.ones_like(pn)

    lhs = jnp.concatenate([p, pn, ones], axis=1)
    rhs = jnp.concatenate([-2.0 * t, ones, tn], axis=1)

    d2 = jax.lax.dot_general(lhs, rhs, _DN, preferred_element_type=jnp.float32)

    fwd = jnp.min(d2, axis=2)
    bwd = jnp.min(d2, axis=1)
    step = jnp.sum(fwd) + jnp.sum(bwd)
    out_ref[...] += step.reshape(1, 1)


def _dir_min_sum(a_ref, b_ref, i, acc):
    ax = [a_ref[i, 0, pl.ds(c * _L, _L)] for c in range(4)]
    ay = [a_ref[i, 1, pl.ds(c * _L, _L)] for c in range(4)]
    az = [a_ref[i, 2, pl.ds(c * _L, _L)] for c in range(4)]
    an = [ax[c] * ax[c] + ay[c] * ay[c] + az[c] * az[c] for c in range(4)]

    big = jnp.full((_L,), _BIG, jnp.float32)
    mins = [[big] * 4, [big] * 4]

    for qc in range(4):
        bx = b_ref[i, 0, pl.ds(qc * _L, _L)]
        by = b_ref[i, 1, pl.ds(qc * _L, _L)]
        bz = b_ref[i, 2, pl.ds(qc * _L, _L)]
        bn = bx * bx + by * by + bz * bz
        bm2x = -2.0 * bx
        bm2y = -2.0 * by
        bm2z = -2.0 * bz
        m = mins[qc & 1]
        for l in range(_L):
            sx, sy, sz, sn = bm2x[l], bm2y[l], bm2z[l], bn[l]
            for c in range(4):
                d2 = (an[c] + sn) + sx * ax[c] + sy * ay[c] + sz * az[c]
                m[c] = jnp.minimum(m[c], d2)

    tot = acc
    for c in range(4):
        tot = tot + jnp.maximum(jnp.minimum(mins[0][c], mins[1][c]), 0.0)
    return tot


def _sc_chamfer(pred_hbm, tgt_hbm, out_hbm, pred_v, tgt_v, acc_v):
    wid = lax.axis_index("s") * _NC + lax.axis_index("c")
    base = _NTC + wid * _PPW
    pltpu.sync_copy(pred_hbm.at[pl.ds(base, _PPW)], pred_v)
    pltpu.sync_copy(tgt_hbm.at[pl.ds(base, _PPW)], tgt_v)

    def patch_body(i, acc):
        acc = _dir_min_sum(pred_v, tgt_v, i, acc)
        acc = _dir_min_sum(tgt_v, pred_v, i, acc)
        return acc

    acc = lax.fori_loop(0, _PPW, patch_body, jnp.zeros((_L,), jnp.float32))
    acc_v[...] = acc
    pltpu.sync_copy(acc_v, out_hbm.at[wid])


def kernel(pred_patches, target_patches):
    pred = pred_patches.reshape(_NP, _P, 3).swapaxes(1, 2)
    tgt = target_patches.reshape(_NP, _P, 3).swapaxes(1, 2)

    mesh = plsc.VectorSubcoreMesh(core_axis_name="c", subcore_axis_name="s")
    sc_run = functools.partial(
        pl.kernel,
        mesh=mesh,
        out_type=jax.ShapeDtypeStruct((_NW, _L), jnp.float32),
        scratch_types=[
            pltpu.VMEM((_PPW, 3, _P), jnp.float32),
            pltpu.VMEM((_PPW, 3, _P), jnp.float32),
            pltpu.VMEM((_L,), jnp.float32),
        ],
    )(_sc_chamfer)
    sc_partial = sc_run(pred, tgt)

    raw = pl.BlockSpec((_BM, 3, _P), lambda i: (i, 0, 0))
    tc_total = pl.pallas_call(
        _chamfer_body,
        grid=(_NTC // _BM,),
        in_specs=[raw, raw],
        out_specs=pl.BlockSpec((1, 1), lambda i: (0, 0)),
        out_shape=jax.ShapeDtypeStruct((1, 1), jnp.float32),
    )(pred, tgt)

    return (tc_total[0, 0] + jnp.sum(sc_partial)) * (1.0 / (_NP * _P))


# speedup vs baseline: 1.1865x; 1.0169x over previous
import functools

import jax
import jax.numpy as jnp
from jax import lax
from jax.experimental import pallas as pl
from jax.experimental.pallas import tpu as pltpu
from jax.experimental.pallas import tpu_sc as plsc

_NP = 4096
_P = 64

_BM = 256
_NTC = 13 * _BM

_L = 16
_NC = 2
_NS = 16
_NW = _NC * _NS
_NSC = _NP - _NTC
_PPW = _NSC // _NW

_BIG = 3.0e38

_DN = (((1,), (1,)), ((0,), (0,)))


def _chamfer_body(pred_ref, tgt_ref, out_ref):
    @pl.when(pl.program_id(0) == 0)
    def _init():
        out_ref[...] = jnp.zeros_like(out_ref)

    p = pred_ref[...]
    t = tgt_ref[...]
    pn = jnp.sum(p * p, axis=1, keepdims=True)
    tn = jnp.sum(t * t, axis=1, keepdims=True)
    ones = jnp.ones_like(pn)

    lhs = jnp.concatenate([p, pn, ones], axis=1)
    rhs = jnp.concatenate([-2.0 * t, ones, tn], axis=1)

    d2 = jax.lax.dot_general(lhs, rhs, _DN, preferred_element_type=jnp.float32)

    fwd = jnp.min(d2, axis=2)
    bwd = jnp.min(d2, axis=1)
    step = jnp.sum(fwd) + jnp.sum(bwd)
    out_ref[...] += step.reshape(1, 1)


def _dir_min_sum(a_ref, b_ref, i, acc):
    ax = [a_ref[i, 0, pl.ds(c * _L, _L)] for c in range(4)]
    ay = [a_ref[i, 1, pl.ds(c * _L, _L)] for c in range(4)]
    az = [a_ref[i, 2, pl.ds(c * _L, _L)] for c in range(4)]
    an = [ax[c] * ax[c] + ay[c] * ay[c] + az[c] * az[c] for c in range(4)]

    big = jnp.full((_L,), _BIG, jnp.float32)
    mins = [[big] * 4, [big] * 4]

    for qc in range(4):
        bx = b_ref[i, 0, pl.ds(qc * _L, _L)]
        by = b_ref[i, 1, pl.ds(qc * _L, _L)]
        bz = b_ref[i, 2, pl.ds(qc * _L, _L)]
        bn = bx * bx + by * by + bz * bz
        bm2x = -2.0 * bx
        bm2y = -2.0 * by
        bm2z = -2.0 * bz
        m = mins[qc & 1]
        for l in range(_L):
            sx, sy, sz, sn = bm2x[l], bm2y[l], bm2z[l], bn[l]
            for c in range(4):
                d2 = (an[c] + sn) + sx * ax[c] + sy * ay[c] + sz * az[c]
                m[c] = jnp.minimum(m[c], d2)

    tot = acc
    for c in range(4):
        tot = tot + jnp.maximum(jnp.minimum(mins[0][c], mins[1][c]), 0.0)
    return tot


def _sc_chamfer(pred_hbm, tgt_hbm, out_hbm, pred_v, tgt_v, acc_v):
    wid = lax.axis_index("s") * _NC + lax.axis_index("c")
    base = _NTC + wid * _PPW
    pltpu.sync_copy(pred_hbm.at[pl.ds(base, _PPW)], pred_v)
    pltpu.sync_copy(tgt_hbm.at[pl.ds(base, _PPW)], tgt_v)

    def patch_body(i, acc):
        acc = _dir_min_sum(pred_v, tgt_v, i, acc)
        acc = _dir_min_sum(tgt_v, pred_v, i, acc)
        return acc

    acc = lax.fori_loop(0, _PPW, patch_body, jnp.zeros((_L,), jnp.float32))
    acc_v[...] = acc
    pltpu.sync_copy(acc_v, out_hbm.at[wid])


def kernel(pred_patches, target_patches):
    pred = pred_patches.reshape(_NP, _P, 3).swapaxes(1, 2)
    tgt = target_patches.reshape(_NP, _P, 3).swapaxes(1, 2)

    mesh = plsc.VectorSubcoreMesh(core_axis_name="c", subcore_axis_name="s")
    sc_run = functools.partial(
        pl.kernel,
        mesh=mesh,
        out_type=jax.ShapeDtypeStruct((_NW, _L), jnp.float32),
        scratch_types=[
            pltpu.VMEM((_PPW, 3, _P), jnp.float32),
            pltpu.VMEM((_PPW, 3, _P), jnp.float32),
            pltpu.VMEM((_L,), jnp.float32),
        ],
    )(_sc_chamfer)
    sc_partial = sc_run(pred, tgt)

    raw = pl.BlockSpec((_BM, 3, _P), lambda i: (i, 0, 0))
    tc_total = pl.pallas_call(
        _chamfer_body,
        grid=(_NTC // _BM,),
        in_specs=[raw, raw],
        out_specs=pl.BlockSpec((1, 1), lambda i: (0, 0)),
        out_shape=jax.ShapeDtypeStruct((1, 1), jnp.float32),
    )(pred, tgt)

    return (tc_total[0, 0] + jnp.sum(sc_partial)) * (1.0 / (_NP * _P))
